# baseline (speedup 1.0000x reference)
import functools

import jax
import jax.numpy as jnp
from jax import lax
from jax.experimental import pallas as pl
from jax.experimental.pallas import tpu as pltpu
from jax.experimental.pallas import tpu_sc as plsc

EMB_DIM = 32
LANES = 16

_info = plsc.get_sparse_core_info()
_NC, _NS = _info.num_cores, _info.num_subcores
_NW = _NC * _NS

_CHUNK = 128
_K = 4


@functools.cache
def _make_gather(B: int):
    b_per_w = B // _NW
    nchunk = b_per_w // _CHUNK
    ngroup = nchunk // _K
    assert B % _NW == 0 and b_per_w % _CHUNK == 0 and nchunk % _K == 0

    mesh = plsc.VectorSubcoreMesh(core_axis_name="c", subcore_axis_name="s")

    @functools.partial(
        pl.kernel,
        mesh=mesh,
        out_type=jax.ShapeDtypeStruct((B * EMB_DIM,), jnp.float32),
        scratch_types=(
            [pltpu.VMEM((b_per_w,), jnp.int32)]
            + [pltpu.VMEM((_CHUNK,), jnp.int32) for _ in range(_K)]
            + [pltpu.VMEM((_CHUNK, 4 * EMB_DIM), jnp.float32)
               for _ in range(_K)]
            + [pltpu.VMEM((_CHUNK * EMB_DIM,), jnp.float32)
               for _ in range(_K)]
            + [pltpu.SemaphoreType.DMA for _ in range(_K)]
        ),
        compiler_params=pltpu.CompilerParams(
            use_tc_tiling_on_sc=True, needs_layout_passes=False),
    )
    def gather_kernel(keys_hbm, table_hbm, out_hbm, idx_v, *bufs):
        gidx_v = bufs[:_K]
        rows_v = bufs[_K:2 * _K]
        stage_v = bufs[2 * _K:3 * _K]
        gsem = bufs[3 * _K:4 * _K]
        wid = lax.axis_index("s") * _NC + lax.axis_index("c")
        base = wid * b_per_w
        pltpu.sync_copy(keys_hbm.at[pl.ds(base, b_per_w)], idx_v)

        lane = lax.iota(jnp.int32, LANES)

        def start_gather(c, b):
            for g in range(_CHUNK // LANES):
                k16 = idx_v[pl.ds(c * _CHUNK + g * LANES, LANES)]
                gidx_v[b][pl.ds(g * LANES, LANES)] = lax.shift_right_logical(
                    k16, jnp.int32(2))
            return pltpu.async_copy(
                table_hbm.at[gidx_v[b]], rows_v[b], gsem[b])

        def extract(c, b):
            for g in range(_CHUNK // LANES):
                k16 = idx_v[pl.ds(c * _CHUNK + g * LANES, LANES)]
                col0 = lax.shift_left(
                    lax.bitwise_and(k16, jnp.int32(3)), jnp.int32(5))
                row = lane + jnp.int32(g * LANES)
                dst0 = lax.mul(row, jnp.int32(EMB_DIM))
                for w in range(EMB_DIM):
                    vals = plsc.load_gather(
                        rows_v[b], [row, col0 + jnp.int32(w)])
                    plsc.store_scatter(
                        stage_v[b], [dst0 + jnp.int32(w)], vals)

        def group(g, carry):
            copies = []
            for b in range(_K):
                copies.append(start_gather(g * _K + b, b))
            for b in range(_K):
                c = g * _K + b
                copies[b].wait()
                extract(c, b)
                pltpu.sync_copy(
                    stage_v[b],
                    out_hbm.at[pl.ds((base + c * _CHUNK) * EMB_DIM,
                                     _CHUNK * EMB_DIM)],
                )
            return carry

        lax.fori_loop(0, ngroup, group, 0)

    return gather_kernel


def kernel(keys, table):
    batch, fields = keys.shape
    B = batch * fields
    kflat = keys.reshape(-1).astype(jnp.int32)
    table4 = table.reshape(table.shape[0] // 4, 4 * EMB_DIM)
    out = _make_gather(B)(kflat, table4)
    return out.reshape(batch, fields, EMB_DIM)


# speedup vs baseline: 1.2423x; 1.2423x over previous
import functools

import jax
import jax.numpy as jnp
from jax import lax
from jax.experimental import pallas as pl
from jax.experimental.pallas import tpu as pltpu
from jax.experimental.pallas import tpu_sc as plsc

EMB_DIM = 32
LANES = 16

_info = plsc.get_sparse_core_info()
_NC, _NS = _info.num_cores, _info.num_subcores
_NW = _NC * _NS

_VBLK = 128
_TK = 4
_CHUNK = 256
_K = 13


@functools.cache
def _make_transpose(V: int, D: int):
    assert D == 2 * LANES
    nfull = V // _VBLK
    tail = V - nfull * _VBLK
    base_cnt = nfull // _NW
    extra = nfull - base_cnt * _NW

    mesh = plsc.VectorSubcoreMesh(core_axis_name="c", subcore_axis_name="s")

    @functools.partial(
        pl.kernel,
        mesh=mesh,
        out_type=jax.ShapeDtypeStruct((V * D,), jnp.float32),
        scratch_types=(
            [pltpu.VMEM((D, _VBLK), jnp.float32) for _ in range(_TK)]
            + [pltpu.VMEM((_VBLK * 2 * LANES,), jnp.float32)
               for _ in range(_TK)]
            + [pltpu.SemaphoreType.DMA for _ in range(_TK)]
            + [pltpu.VMEM((D, _VBLK), jnp.float32),
               pltpu.VMEM((_VBLK * 2 * LANES,), jnp.float32),
               pltpu.VMEM((_VBLK, D), jnp.float32)]
        ),
        compiler_params=pltpu.CompilerParams(
            use_tc_tiling_on_sc=True, needs_layout_passes=False),
    )
    def transpose_kernel(tab_t_hbm, tail_hbm, out_hbm, *bufs):
        src_v = bufs[:_TK]
        stg_v = bufs[_TK:2 * _TK]
        rsem = bufs[2 * _TK:3 * _TK]
        tsrc_v = bufs[3 * _TK]
        tstg_v = bufs[3 * _TK + 1]
        tail_v = bufs[3 * _TK + 2]
        wid = lax.axis_index("s") * _NC + lax.axis_index("c")

        lane = lax.iota(jnp.int32, LANES)
        row_lo = lane
        row_hi = lane + jnp.int32(LANES)

        def transpose_block(src, stg, nv):
            for v in range(nv):
                col = jnp.full((LANES,), v, jnp.int32)
                lo = plsc.load_gather(src, [row_lo, col])
                hi = plsc.load_gather(src, [row_hi, col])
                stg[pl.ds(v * EMB_DIM, LANES)] = lo
                stg[pl.ds(v * EMB_DIM + LANES, LANES)] = hi

        def start_read(blk, b):
            return pltpu.async_copy(
                tab_t_hbm.at[:, pl.ds(blk * _VBLK, _VBLK)], src_v[b], rsem[b])

        ngroup_full = base_cnt // _TK
        assert base_cnt % _TK == 0

        def group(g, carry):
            copies = []
            for b in range(_TK):
                j = g * _TK + b
                copies.append(start_read(wid + j * _NW, b))
            for b in range(_TK):
                j = g * _TK + b
                copies[b].wait()
                transpose_block(src_v[b], stg_v[b], _VBLK)
                pltpu.sync_copy(
                    stg_v[b],
                    out_hbm.at[pl.ds((wid + j * _NW) * _VBLK * D, _VBLK * D)],
                )
            return carry

        lax.fori_loop(0, ngroup_full, group, 0)

        @pl.when(wid < extra)
        def _():
            blk = base_cnt * _NW + wid
            pltpu.async_copy(
                tab_t_hbm.at[:, pl.ds(blk * _VBLK, _VBLK)], tsrc_v,
                rsem[0]).wait()
            transpose_block(tsrc_v, tstg_v, _VBLK)
            pltpu.sync_copy(
                tstg_v, out_hbm.at[pl.ds(blk * _VBLK * D, _VBLK * D)])

        if tail:
            @pl.when(wid == _NW - 1)
            def _():
                v0 = nfull * _VBLK
                pltpu.sync_copy(tail_hbm.at[pl.ds(0, tail)],
                                tail_v.at[pl.ds(0, tail)])
                for r in range(tail):
                    tstg_v[pl.ds(r * D, LANES)] = tail_v[r, pl.ds(0, LANES)]
                    tstg_v[pl.ds(r * D + LANES, LANES)] = (
                        tail_v[r, pl.ds(LANES, LANES)])
                pltpu.sync_copy(
                    tstg_v.at[pl.ds(0, tail * D)],
                    out_hbm.at[pl.ds(v0 * D, tail * D)])

    return transpose_kernel


@functools.cache
def _make_gather(B: int):
    b_per_w = B // _NW
    nchunk = b_per_w // _CHUNK
    ngroup = nchunk // _K
    assert B % _NW == 0 and b_per_w % _CHUNK == 0 and nchunk % _K == 0

    mesh = plsc.VectorSubcoreMesh(core_axis_name="c", subcore_axis_name="s")

    @functools.partial(
        pl.kernel,
        mesh=mesh,
        out_type=jax.ShapeDtypeStruct((B, EMB_DIM), jnp.float32),
        scratch_types=[
            pltpu.VMEM((b_per_w,), jnp.int32),
            pltpu.VMEM((_K, _CHUNK, EMB_DIM), jnp.float32),
            pltpu.SemaphoreType.DMA((_K,)),
        ],
        compiler_params=pltpu.CompilerParams(use_tc_tiling_on_sc=False),
    )
    def gather_kernel(keys_hbm, table_hbm, out_hbm, idx_v, rows_v, gsem):
        wid = lax.axis_index("s") * _NC + lax.axis_index("c")
        base = wid * b_per_w
        pltpu.sync_copy(keys_hbm.at[pl.ds(base, b_per_w)], idx_v)

        def group(g, carry):
            copies = []
            for b in range(_K):
                c = g * _K + b
                copies.append(
                    pltpu.async_copy(
                        table_hbm.at[idx_v.at[pl.ds(c * _CHUNK, _CHUNK)]],
                        rows_v.at[b],
                        gsem.at[b],
                    )
                )
            for b in range(_K):
                c = g * _K + b
                copies[b].wait()
                pltpu.sync_copy(
                    rows_v.at[b],
                    out_hbm.at[pl.ds(base + c * _CHUNK, _CHUNK)],
                )
            return carry

        lax.fori_loop(0, ngroup, group, 0)

    return gather_kernel


def kernel(keys, table):
    batch, fields = keys.shape
    B = batch * fields
    V, D = table.shape
    kflat = keys.reshape(-1).astype(jnp.int32)
    ntail = V % _VBLK
    tail_rows = lax.slice(table, (V - max(ntail, 1), 0), (V, D))
    tab_flat = _make_transpose(V, D)(table.T, tail_rows)
    out = _make_gather(B)(kflat, tab_flat.reshape(V, D))
    return out.reshape(batch, fields, EMB_DIM)


# speedup vs baseline: 1.3336x; 1.0735x over previous
import functools

import jax
import jax.numpy as jnp
from jax import lax
from jax.experimental import pallas as pl
from jax.experimental.pallas import tpu as pltpu
from jax.experimental.pallas import tpu_sc as plsc

EMB_DIM = 32
LANES = 16

_info = plsc.get_sparse_core_info()
_NC, _NS = _info.num_cores, _info.num_subcores
_NW = _NC * _NS

_VBLK = 128
_TK = 4
_CHUNK = 256
_K = 13


@functools.cache
def _make_transpose(V: int, D: int):
    assert D == 2 * LANES
    nfull = V // _VBLK
    tail = V - nfull * _VBLK
    base_cnt = nfull // _NW
    extra = nfull - base_cnt * _NW

    mesh = plsc.VectorSubcoreMesh(core_axis_name="c", subcore_axis_name="s")

    @functools.partial(
        pl.kernel,
        mesh=mesh,
        out_type=jax.ShapeDtypeStruct((V * D,), jnp.float32),
        scratch_types=(
            [pltpu.VMEM((D, _VBLK), jnp.float32) for _ in range(_TK)]
            + [pltpu.VMEM((_VBLK * 2 * LANES,), jnp.float32)
               for _ in range(_TK)]
            + [pltpu.SemaphoreType.DMA for _ in range(_TK)]
            + [pltpu.SemaphoreType.DMA for _ in range(_TK)]
            + [pltpu.VMEM((D, _VBLK), jnp.float32),
               pltpu.VMEM((_VBLK * 2 * LANES,), jnp.float32),
               pltpu.VMEM((_VBLK, D), jnp.float32)]
        ),
        compiler_params=pltpu.CompilerParams(
            use_tc_tiling_on_sc=True, needs_layout_passes=False),
    )
    def transpose_kernel(tab_t_hbm, tail_hbm, out_hbm, *bufs):
        src_v = bufs[:_TK]
        stg_v = bufs[_TK:2 * _TK]
        rsem = bufs[2 * _TK:3 * _TK]
        wsem = bufs[3 * _TK:4 * _TK]
        tsrc_v = bufs[4 * _TK]
        tstg_v = bufs[4 * _TK + 1]
        tail_v = bufs[4 * _TK + 2]
        wid = lax.axis_index("s") * _NC + lax.axis_index("c")

        lane = lax.iota(jnp.int32, LANES)
        row_lo = lane
        row_hi = lane + jnp.int32(LANES)

        def transpose_block(src, stg, nv):
            for vg in range(0, nv, 8):
                vals = []
                for dv in range(8):
                    col = jnp.full((LANES,), vg + dv, jnp.int32)
                    vals.append(plsc.load_gather(src, [row_lo, col]))
                    vals.append(plsc.load_gather(src, [row_hi, col]))
                for dv in range(8):
                    v = vg + dv
                    stg[pl.ds(v * EMB_DIM, LANES)] = vals[2 * dv]
                    stg[pl.ds(v * EMB_DIM + LANES, LANES)] = vals[2 * dv + 1]

        def start_read(blk, b):
            return pltpu.async_copy(
                tab_t_hbm.at[:, pl.ds(blk * _VBLK, _VBLK)], src_v[b], rsem[b])

        ngroup_full = base_cnt // _TK
        assert base_cnt % _TK == 0

        def group(g, carry):
            copies = []
            for b in range(_TK):
                j = g * _TK + b
                copies.append(start_read(wid + j * _NW, b))
            wcopies = []
            for b in range(_TK):
                j = g * _TK + b
                copies[b].wait()
                transpose_block(src_v[b], stg_v[b], _VBLK)
                wcopies.append(pltpu.async_copy(
                    stg_v[b],
                    out_hbm.at[pl.ds((wid + j * _NW) * _VBLK * D, _VBLK * D)],
                    wsem[b],
                ))
            for w in wcopies:
                w.wait()
            return carry

        lax.fori_loop(0, ngroup_full, group, 0)

        @pl.when(wid < extra)
        def _():
            blk = base_cnt * _NW + wid
            pltpu.async_copy(
                tab_t_hbm.at[:, pl.ds(blk * _VBLK, _VBLK)], tsrc_v,
                rsem[0]).wait()
            transpose_block(tsrc_v, tstg_v, _VBLK)
            pltpu.sync_copy(
                tstg_v, out_hbm.at[pl.ds(blk * _VBLK * D, _VBLK * D)])

        if tail:
            @pl.when(wid == _NW - 1)
            def _():
                v0 = nfull * _VBLK
                pltpu.sync_copy(tail_hbm.at[pl.ds(0, tail)],
                                tail_v.at[pl.ds(0, tail)])
                for r in range(tail):
                    tstg_v[pl.ds(r * D, LANES)] = tail_v[r, pl.ds(0, LANES)]
                    tstg_v[pl.ds(r * D + LANES, LANES)] = (
                        tail_v[r, pl.ds(LANES, LANES)])
                pltpu.sync_copy(
                    tstg_v.at[pl.ds(0, tail * D)],
                    out_hbm.at[pl.ds(v0 * D, tail * D)])

    return transpose_kernel


@functools.cache
def _make_gather(B: int):
    b_per_w = B // _NW
    nchunk = b_per_w // _CHUNK
    ngroup = nchunk // _K
    assert B % _NW == 0 and b_per_w % _CHUNK == 0 and nchunk % _K == 0

    mesh = plsc.VectorSubcoreMesh(core_axis_name="c", subcore_axis_name="s")

    @functools.partial(
        pl.kernel,
        mesh=mesh,
        out_type=jax.ShapeDtypeStruct((B, EMB_DIM), jnp.float32),
        scratch_types=[
            pltpu.VMEM((b_per_w,), jnp.int32),
            pltpu.VMEM((_K, _CHUNK, EMB_DIM), jnp.float32),
            pltpu.SemaphoreType.DMA((_K,)),
        ],
        compiler_params=pltpu.CompilerParams(use_tc_tiling_on_sc=False),
    )
    def gather_kernel(keys_hbm, table_hbm, out_hbm, idx_v, rows_v, gsem):
        wid = lax.axis_index("s") * _NC + lax.axis_index("c")
        base = wid * b_per_w
        pltpu.sync_copy(keys_hbm.at[pl.ds(base, b_per_w)], idx_v)

        def group(g, carry):
            copies = []
            for b in range(_K):
                c = g * _K + b
                copies.append(
                    pltpu.async_copy(
                        table_hbm.at[idx_v.at[pl.ds(c * _CHUNK, _CHUNK)]],
                        rows_v.at[b],
                        gsem.at[b],
                    )
                )
            for b in range(_K):
                c = g * _K + b
                copies[b].wait()
                pltpu.sync_copy(
                    rows_v.at[b],
                    out_hbm.at[pl.ds(base + c * _CHUNK, _CHUNK)],
                )
            return carry

        lax.fori_loop(0, ngroup, group, 0)

    return gather_kernel


def kernel(keys, table):
    batch, fields = keys.shape
    B = batch * fields
    V, D = table.shape
    kflat = keys.reshape(-1).astype(jnp.int32)
    ntail = V % _VBLK
    tail_rows = lax.slice(table, (V - max(ntail, 1), 0), (V, D))
    tab_flat = _make_transpose(V, D)(table.T, tail_rows)
    out = _make_gather(B)(kflat, tab_flat.reshape(V, D))
    return out.reshape(batch, fields, EMB_DIM)


# speedup vs baseline: 1.7726x; 1.3291x over previous
import functools

import jax
import jax.numpy as jnp
from jax import lax
from jax.experimental import pallas as pl
from jax.experimental.pallas import tpu as pltpu
from jax.experimental.pallas import tpu_sc as plsc

EMB_DIM = 32

_info = plsc.get_sparse_core_info()
_NC, _NS = _info.num_cores, _info.num_subcores
_NW = _NC * _NS

_CHUNK = 256
_K = 13


@functools.cache
def _make_gather(B: int):
    b_per_w = B // _NW
    nchunk = b_per_w // _CHUNK
    ngroup = nchunk // _K
    assert B % _NW == 0 and b_per_w % _CHUNK == 0 and nchunk % _K == 0

    mesh = plsc.VectorSubcoreMesh(core_axis_name="c", subcore_axis_name="s")

    @functools.partial(
        pl.kernel,
        mesh=mesh,
        out_type=jax.ShapeDtypeStruct((B, EMB_DIM), jnp.float32),
        scratch_types=[
            pltpu.VMEM((nchunk, _CHUNK), jnp.int32),
            pltpu.VMEM((_K, _CHUNK, EMB_DIM), jnp.float32),
            pltpu.SemaphoreType.DMA((_K,)),
        ],
        compiler_params=pltpu.CompilerParams(use_tc_tiling_on_sc=False),
    )
    def gather_kernel(keys_hbm, table_hbm, out_hbm, idx_v, rows_v, gsem):
        wid = lax.axis_index("s") * _NC + lax.axis_index("c")
        base = wid * b_per_w
        pltpu.sync_copy(keys_hbm.at[wid], idx_v)

        def group(g, carry):
            copies = []
            for b in range(_K):
                c = g * _K + b
                copies.append(
                    pltpu.async_copy(
                        table_hbm.at[idx_v.at[c]], rows_v.at[b], gsem.at[b]
                    )
                )
            for b in range(_K):
                c = g * _K + b
                copies[b].wait()
                pltpu.sync_copy(
                    rows_v.at[b],
                    out_hbm.at[pl.ds(base + c * _CHUNK, _CHUNK)],
                )
            return carry

        lax.fori_loop(0, ngroup, group, 0)

    return gather_kernel


def kernel(keys, table):
    batch, fields = keys.shape
    B = batch * fields
    b_per_w = B // _NW
    nchunk = b_per_w // _CHUNK
    karr = keys.reshape(-1).astype(jnp.int32).reshape(_NW, nchunk, _CHUNK)
    out = _make_gather(B)(karr, table)
    return out.reshape(batch, fields, EMB_DIM)
